# baseline (speedup 1.0000x reference)
import functools

import jax
import jax.numpy as jnp
import numpy as np
from jax import lax
from jax.experimental import pallas as pl
from jax.experimental.pallas import tpu as pltpu
from jax.experimental.pallas import tpu_sc as plsc

N = 10000
E = 320000
NPAD = 10240
NC, NS, LANES = 2, 16, 16
B = 64
ET = E + N
NB = -(-ET // (NC * NS * B))
ETP = NC * NS * NB * B

_COLPERM = np.array(
    [16 * ((c % 16) // 2) + 2 * (c // 16) + (c % 16) % 2 for c in range(128)],
    dtype=np.int32,
)

_R = 512
_GRID = NPAD // _R


def _fold_att(W, att):
    heads, ch = att.shape[1], att.shape[2]
    return jnp.einsum("ihc,hc->ih", W.reshape(W.shape[0], heads, ch), att[0])


def _prep_body(cm, x_ref, w_ref, h_ref, as_ref, ad_ref, m_ref):
    i = pl.program_id(0)
    h = jnp.dot(x_ref[...], w_ref[...], preferred_element_type=jnp.float32)
    h_ref[:, :cm] = h[:, :cm]
    h_ref[:, cm:] = jnp.ones((_R, 16), jnp.float32)
    a_s = h[:, cm:cm + 16]
    a_d = h[:, cm + 16:cm + 32]
    as_ref[...] = a_s
    ad_ref[...] = a_d
    m = jnp.max(a_s, axis=0, keepdims=True) + jnp.max(a_d, axis=0, keepdims=True)

    @pl.when(i == 0)
    def _():
        m_ref[...] = m

    @pl.when(i > 0)
    def _():
        m_ref[...] = jnp.maximum(m_ref[...], m)


@functools.partial(jax.jit, static_argnums=(2,))
def _prep(xp, wcat, cm):
    cw = cm + 32
    ct = cm + 16
    return pl.pallas_call(
        functools.partial(_prep_body, cm),
        grid=(_GRID,),
        in_specs=[
            pl.BlockSpec((_R, 128), lambda i: (i, 0)),
            pl.BlockSpec((128, cw), lambda i: (0, 0)),
        ],
        out_specs=[
            pl.BlockSpec((_R, ct), lambda i: (i, 0)),
            pl.BlockSpec((_R, 16), lambda i: (i, 0)),
            pl.BlockSpec((_R, 16), lambda i: (i, 0)),
            pl.BlockSpec((1, 16), lambda i: (0, 0)),
        ],
        out_shape=[
            jax.ShapeDtypeStruct((NPAD, ct), jnp.float32),
            jax.ShapeDtypeStruct((NPAD, 16), jnp.float32),
            jax.ShapeDtypeStruct((NPAD, 16), jnp.float32),
            jax.ShapeDtypeStruct((1, 16), jnp.float32),
        ],
    )(xp, wcat)


def _edge_body(ct, h_hbm, as_hbm, ad_hbm, m_hbm, idx_hbm, out_hbm,
               acc, idx_v, as_v, ad_v, h_v, m_v,
               gsa, gsd, gsh, scs, ixs):
    nq = ct // 16
    c = lax.axis_index("c")
    s = lax.axis_index("s")
    wid = c * NS + s
    rows = NPAD // NS

    def g_as(b_slot, i_slot):
        return pltpu.make_async_copy(
            as_hbm.at[idx_v.at[i_slot, 0]], as_v.at[b_slot], gsa.at[b_slot])

    def g_ad(b_slot, i_slot):
        return pltpu.make_async_copy(
            ad_hbm.at[idx_v.at[i_slot, 1]], ad_v.at[b_slot], gsd.at[b_slot])

    def g_h(b_slot, i_slot):
        return pltpu.make_async_copy(
            h_hbm.at[idx_v.at[i_slot, 0]], h_v.at[b_slot], gsh.at[b_slot])

    def sc_start(b_slot, i_slot):
        pltpu.async_copy(h_v.at[b_slot], acc.at[idx_v.at[i_slot, 1]],
                         scs.at[b_slot], add=True)

    def sc_wait(b_slot):
        pltpu.make_async_copy(h_v.at[b_slot], acc.at[idx_v.at[0, 1]],
                              scs.at[b_slot]).wait()

    def g_idx(blk, i_slot, par):
        return pltpu.make_async_copy(
            idx_hbm.at[wid, blk], idx_v.at[i_slot], ixs.at[par])

    def issue_gather(b_slot, i_slot):
        g_as(b_slot, i_slot).start()
        g_ad(b_slot, i_slot).start()
        g_h(b_slot, i_slot).start()

    def wait_gather(b_slot):
        g_as(b_slot, 0).wait()
        g_ad(b_slot, 0).wait()
        g_h(b_slot, 0).wait()

    def zfill(r, _):
        for q in range(nq):
            h_v[0, r, pl.ds(16 * q, 16)] = jnp.zeros((16,), jnp.float32)
        return 0
    lax.fori_loop(0, B, zfill, 0)

    def zcopy(k, _):
        pltpu.sync_copy(h_v.at[0], acc.at[pl.ds(s * rows + k * B, B)])
        return 0
    lax.fori_loop(0, rows // B, zcopy, 0)

    pltpu.sync_copy(m_hbm, m_v)
    pltpu.sync_copy(idx_hbm.at[wid, 0], idx_v.at[0])
    pltpu.sync_copy(idx_hbm.at[wid, 1], idx_v.at[1])
    g_idx(2, 2, 0).start()
    plsc.subcore_barrier()
    issue_gather(0, 0)
    issue_gather(1, 1)
    mvec = m_v[...]

    def blk(b, _):
        s0 = lax.rem(b, 3)
        s2 = lax.rem(b + 2, 3)
        i0 = lax.rem(b, 4)
        i2 = lax.rem(b + 2, 4)
        i3 = lax.rem(b + 3, 4)
        wait_gather(s0)

        def edge(i, _):
            e = as_v[s0, i, :] + ad_v[s0, i, :]
            e = jnp.maximum(e, 0.2 * e)
            w = jnp.exp(e - mvec)
            for q in range(nq):
                h_v[s0, i, pl.ds(16 * q, 16)] = (
                    h_v[s0, i, pl.ds(16 * q, 16)] * w)
            return 0
        lax.fori_loop(0, B, edge, 0)
        sc_start(s0, i0)

        @pl.when(jnp.logical_and(b >= 1, b + 2 < NB))
        def _():
            sc_wait(s2)

        @pl.when(b + 3 < NB)
        def _():
            g_idx(b + 3, i3, lax.rem(b + 3, 2)).start()

        @pl.when(b + 2 < NB)
        def _():
            g_idx(0, 0, lax.rem(b + 2, 2)).wait()
            issue_gather(s2, i2)
        return 0
    lax.fori_loop(0, NB, blk, 0)

    sc_wait(0)
    sc_wait(1)
    sc_wait(2)

    plsc.subcore_barrier()
    pltpu.sync_copy(acc.at[pl.ds(s * rows, rows)],
                    out_hbm.at[c, pl.ds(s * rows, rows)])


@functools.partial(jax.jit, static_argnums=(5,))
def _edge_pass(h_tab, as_tab, ad_tab, m16, idx, ct):
    mesh = plsc.VectorSubcoreMesh(
        core_axis_name="c", subcore_axis_name="s",
        num_cores=NC, num_subcores=NS)
    return pl.kernel(
        functools.partial(_edge_body, ct),
        out_type=jax.ShapeDtypeStruct((NC, NPAD, ct), jnp.float32),
        mesh=mesh,
        compiler_params=pltpu.CompilerParams(use_tc_tiling_on_sc=False),
        scratch_types=[
            pltpu.VMEM_SHARED((NPAD, ct), jnp.float32),
            pltpu.VMEM((4, 2, B), jnp.int32),
            pltpu.VMEM((3, B, 16), jnp.float32),
            pltpu.VMEM((3, B, 16), jnp.float32),
            pltpu.VMEM((3, B, ct), jnp.float32),
            pltpu.VMEM((16,), jnp.float32),
            pltpu.SemaphoreType.DMA((3,)),
            pltpu.SemaphoreType.DMA((3,)),
            pltpu.SemaphoreType.DMA((3,)),
            pltpu.SemaphoreType.DMA((3,)),
            pltpu.SemaphoreType.DMA((2,)),
        ],
    )(h_tab, as_tab, ad_tab, m16, idx)


def _fin_body(cm, acc_ref, b_ref, o_ref):
    a = acc_ref[0] + acc_ref[1]
    den = a[:, cm:cm + 16]
    dfull = jnp.concatenate([den] * (cm // 16), axis=1)
    o = a[:, :cm] / (dfull + 1e-16) + b_ref[...]
    o_ref[...] = jnp.where(o > 0, o, jnp.exp(o) - 1.0)


def _fin3_body(acc_ref, b_ref, o_ref):
    a = acc_ref[0] + acc_ref[1]
    den = a[:, 64:80]
    dfull = jnp.concatenate([den] * 4, axis=1)
    o = a[:, :64] / (dfull + 1e-16) + b_ref[...]
    m = jnp.max(o, axis=1, keepdims=True)
    z = o - m
    o_ref[...] = z - jnp.log(jnp.sum(jnp.exp(z), axis=1, keepdims=True))


@jax.jit
def _finalize(acc, bias):
    ct = acc.shape[2]
    cm = ct - 16
    return pl.pallas_call(
        functools.partial(_fin_body, cm),
        grid=(_GRID,),
        in_specs=[
            pl.BlockSpec((2, _R, ct), lambda i: (0, i, 0)),
            pl.BlockSpec((1, cm), lambda i: (0, 0)),
        ],
        out_specs=pl.BlockSpec((_R, cm), lambda i: (i, 0)),
        out_shape=jax.ShapeDtypeStruct((NPAD, cm), jnp.float32),
    )(acc, bias)


@jax.jit
def _finalize3(acc, bias):
    return pl.pallas_call(
        _fin3_body,
        grid=(_GRID,),
        in_specs=[
            pl.BlockSpec((2, _R, 80), lambda i: (0, i, 0)),
            pl.BlockSpec((1, 64), lambda i: (0, 0)),
        ],
        out_specs=pl.BlockSpec((_R, 64), lambda i: (i, 0)),
        out_shape=jax.ShapeDtypeStruct((NPAD, 64), jnp.float32),
    )(acc, bias)


def _leaky(x):
    return jnp.maximum(x, 0.2 * x)


def kernel(x, edge_index, W1, as1, ad1, b1, W2, as2, ad2, b2, W3, as3, ad3, b3):
    cp = jnp.asarray(_COLPERM)

    loop = jnp.arange(N, dtype=jnp.int32)
    padv = jnp.full((ETP - ET,), N, jnp.int32)
    src = jnp.concatenate([edge_index[0], loop, padv]).reshape(NC * NS, NB, B)
    dst = jnp.concatenate([edge_index[1], loop, padv]).reshape(NC * NS, NB, B)
    idx = jnp.stack([src, dst], axis=2)

    rep2 = lambda a: jnp.repeat(a, 2, axis=1)
    wcat1 = jnp.concatenate(
        [W1[:, cp], rep2(_fold_att(W1, as1)), rep2(_fold_att(W1, ad1))], axis=1)
    W2r = W2[cp, :]
    wcat2 = jnp.concatenate(
        [W2r[:, cp], rep2(_fold_att(W2r, as2)), rep2(_fold_att(W2r, ad2))], axis=1)
    W3r = W3[cp, :]
    rep16 = lambda a: jnp.repeat(a, 16, axis=1)
    wcat3 = jnp.concatenate(
        [W3r, rep16(_fold_att(W3r, as3)), rep16(_fold_att(W3r, ad3))], axis=1)
    b1p = b1[cp][None, :]
    b2p = b2[cp][None, :]
    b3p = b3[None, :]

    xp = jnp.pad(x, ((0, NPAD - N), (0, 0)))

    h_tab, a_s, a_d, mraw = _prep(xp, wcat1, 128)
    m16 = _leaky(mraw[0])
    acc = _edge_pass(h_tab, a_s, a_d, m16, idx, 144)
    x1 = _finalize(acc, b1p)

    h_tab, a_s, a_d, mraw = _prep(x1, wcat2, 128)
    m16 = _leaky(mraw[0])
    acc = _edge_pass(h_tab, a_s, a_d, m16, idx, 144)
    x2 = _finalize(acc, b2p)

    h_tab, a_s, a_d, mraw = _prep(x2, wcat3, 64)
    m16 = _leaky(mraw[0])
    acc = _edge_pass(h_tab, a_s, a_d, m16, idx, 80)
    out = _finalize3(acc, b3p)
    return out[:N]


# speedup vs baseline: 1.3608x; 1.3608x over previous
import functools

import jax
import jax.numpy as jnp
import numpy as np
from jax import lax
from jax.experimental import pallas as pl
from jax.experimental.pallas import tpu as pltpu
from jax.experimental.pallas import tpu_sc as plsc

N = 10000
E = 320000
NPAD = 10240
NACC = 10016
NC, NS, LANES = 2, 16, 16
B = 112
ET = E + N
NB = -(-ET // (NC * NS * B))
ETP = NC * NS * NB * B

_COLPERM = np.array(
    [16 * ((c % 16) // 2) + 2 * (c // 16) + (c % 16) % 2 for c in range(128)],
    dtype=np.int32,
)

_R = 512
_GRID = NPAD // _R


def _fold_att(W, att):
    heads, ch = att.shape[1], att.shape[2]
    return jnp.einsum("ihc,hc->ih", W.reshape(W.shape[0], heads, ch), att[0])


def _prep_body(cm, x_ref, w_ref, h_ref, as_ref, ad_ref, m_ref):
    i = pl.program_id(0)
    h = jnp.dot(x_ref[...], w_ref[...], preferred_element_type=jnp.float32)
    h_ref[:, :cm] = h[:, :cm]
    h_ref[:, cm:] = jnp.ones((_R, 16), jnp.float32)
    a_s = h[:, cm:cm + 16]
    a_d = h[:, cm + 16:cm + 32]
    as_ref[...] = a_s
    ad_ref[...] = a_d
    m = jnp.max(a_s, axis=0, keepdims=True) + jnp.max(a_d, axis=0, keepdims=True)

    @pl.when(i == 0)
    def _():
        m_ref[...] = m

    @pl.when(i > 0)
    def _():
        m_ref[...] = jnp.maximum(m_ref[...], m)


@functools.partial(jax.jit, static_argnums=(2,))
def _prep(xp, wcat, cm):
    cw = cm + 32
    ct = cm + 16
    return pl.pallas_call(
        functools.partial(_prep_body, cm),
        grid=(_GRID,),
        in_specs=[
            pl.BlockSpec((_R, 128), lambda i: (i, 0)),
            pl.BlockSpec((128, cw), lambda i: (0, 0)),
        ],
        out_specs=[
            pl.BlockSpec((_R, ct), lambda i: (i, 0)),
            pl.BlockSpec((_R, 16), lambda i: (i, 0)),
            pl.BlockSpec((_R, 16), lambda i: (i, 0)),
            pl.BlockSpec((1, 16), lambda i: (0, 0)),
        ],
        out_shape=[
            jax.ShapeDtypeStruct((NPAD, ct), jnp.float32),
            jax.ShapeDtypeStruct((NPAD, 16), jnp.float32),
            jax.ShapeDtypeStruct((NPAD, 16), jnp.float32),
            jax.ShapeDtypeStruct((1, 16), jnp.float32),
        ],
    )(xp, wcat)


def _edge_body(ct, h_hbm, as_hbm, ad_hbm, m_hbm, idx_hbm, out_hbm,
               acc, idx_v, as_v, ad_v, h_v, m_v,
               gsa, gsd, gsh, scs, ixs):
    nq = ct // 16
    c = lax.axis_index("c")
    s = lax.axis_index("s")
    wid = c * NS + s
    rows = NACC // NS

    def g_as(b_slot, i_slot):
        return pltpu.make_async_copy(
            as_hbm.at[idx_v.at[i_slot, 0]], as_v.at[b_slot], gsa.at[b_slot])

    def g_ad(b_slot, i_slot):
        return pltpu.make_async_copy(
            ad_hbm.at[idx_v.at[i_slot, 1]], ad_v.at[b_slot], gsd.at[b_slot])

    def g_h(b_slot, i_slot):
        return pltpu.make_async_copy(
            h_hbm.at[idx_v.at[i_slot, 0]], h_v.at[b_slot], gsh.at[b_slot])

    def sc_start(b_slot, i_slot):
        pltpu.async_copy(h_v.at[b_slot], acc.at[idx_v.at[i_slot, 1]],
                         scs, add=True)

    def sc_wait():
        pltpu.make_async_copy(h_v.at[0], acc.at[idx_v.at[0, 1]], scs).wait()

    def g_idx(blk, i_slot):
        return pltpu.make_async_copy(
            idx_hbm.at[wid, blk], idx_v.at[i_slot], ixs)

    def issue_gather(b_slot, i_slot):
        g_as(b_slot, i_slot).start()
        g_ad(b_slot, i_slot).start()
        g_h(b_slot, i_slot).start()

    def wait_gather(b_slot):
        g_as(b_slot, 0).wait()
        g_ad(b_slot, 0).wait()
        g_h(b_slot, 0).wait()

    def zfill(r, _):
        for q in range(nq):
            h_v[0, r, pl.ds(16 * q, 16)] = jnp.zeros((16,), jnp.float32)
        return 0
    lax.fori_loop(0, B, zfill, 0)

    def zcopy(k, _):
        pltpu.sync_copy(h_v.at[0], acc.at[pl.ds(s * rows + k * B, B)])
        return 0
    lax.fori_loop(0, rows // B, zcopy, 0)
    pltpu.sync_copy(h_v.at[0, pl.ds(0, rows - B * (rows // B))],
                    acc.at[pl.ds(s * rows + B * (rows // B),
                                 rows - B * (rows // B))])

    pltpu.sync_copy(m_hbm, m_v)
    pltpu.sync_copy(idx_hbm.at[wid, 0], idx_v.at[0])
    pltpu.sync_copy(idx_hbm.at[wid, 1], idx_v.at[1])
    g_idx(2, 2).start()
    plsc.subcore_barrier()
    issue_gather(0, 0)
    mvec = m_v[...]

    def blk(b, _):
        p = lax.rem(b, 2)
        pbar = 1 - p
        i1 = lax.rem(b + 1, 3)
        i2 = lax.rem(b + 2, 3)
        wait_gather(p)

        @pl.when(b >= 1)
        def _():
            sc_wait()

        @pl.when(jnp.logical_and(b >= 1, b + 1 < NB))
        def _():
            g_idx(0, 0).wait()

        @pl.when(b + 1 < NB)
        def _():
            issue_gather(pbar, i1)

        def edge(i, _):
            e = as_v[p, i, :] + ad_v[p, i, :]
            e = jnp.maximum(e, 0.2 * e)
            w = jnp.exp(e - mvec)
            for q in range(nq):
                h_v[p, i, pl.ds(16 * q, 16)] = (
                    h_v[p, i, pl.ds(16 * q, 16)] * w)
            return 0
        lax.fori_loop(0, B, edge, 0)
        sc_start(p, lax.rem(b, 3))

        @pl.when(jnp.logical_and(b >= 1, b + 2 < NB))
        def _():
            g_idx(b + 2, i2).start()
        return 0
    lax.fori_loop(0, NB, blk, 0)

    sc_wait()
    plsc.subcore_barrier()
    pltpu.sync_copy(acc.at[pl.ds(s * rows, rows)],
                    out_hbm.at[c, pl.ds(s * rows, rows)])


@functools.partial(jax.jit, static_argnums=(5,))
def _edge_pass(h_tab, as_tab, ad_tab, m16, idx, ct):
    mesh = plsc.VectorSubcoreMesh(
        core_axis_name="c", subcore_axis_name="s",
        num_cores=NC, num_subcores=NS)
    return pl.kernel(
        functools.partial(_edge_body, ct),
        out_type=jax.ShapeDtypeStruct((NC, NACC, ct), jnp.float32),
        mesh=mesh,
        compiler_params=pltpu.CompilerParams(use_tc_tiling_on_sc=False),
        scratch_types=[
            pltpu.VMEM_SHARED((NACC, ct), jnp.float32),
            pltpu.VMEM((3, 2, B), jnp.int32),
            pltpu.VMEM((2, B, 16), jnp.float32),
            pltpu.VMEM((2, B, 16), jnp.float32),
            pltpu.VMEM((2, B, ct), jnp.float32),
            pltpu.VMEM((16,), jnp.float32),
            pltpu.SemaphoreType.DMA((2,)),
            pltpu.SemaphoreType.DMA((2,)),
            pltpu.SemaphoreType.DMA((2,)),
            pltpu.SemaphoreType.DMA,
            pltpu.SemaphoreType.DMA,
        ],
    )(h_tab, as_tab, ad_tab, m16, idx)


def _fin_body(cm, acc_ref, b_ref, o_ref):
    a = acc_ref[0] + acc_ref[1]
    den = a[:, cm:cm + 16]
    dfull = jnp.concatenate([den] * (cm // 16), axis=1)
    o = a[:, :cm] / (dfull + 1e-16) + b_ref[...]
    o_ref[...] = jnp.where(o > 0, o, jnp.exp(o) - 1.0)


def _fin3_body(acc_ref, b_ref, o_ref):
    a = acc_ref[0] + acc_ref[1]
    den = a[:, 64:80]
    dfull = jnp.concatenate([den] * 4, axis=1)
    o = a[:, :64] / (dfull + 1e-16) + b_ref[...]
    m = jnp.max(o, axis=1, keepdims=True)
    z = o - m
    o_ref[...] = z - jnp.log(jnp.sum(jnp.exp(z), axis=1, keepdims=True))


@jax.jit
def _finalize(acc, bias):
    ct = acc.shape[2]
    cm = ct - 16
    return pl.pallas_call(
        functools.partial(_fin_body, cm),
        out_shape=jax.ShapeDtypeStruct((NACC, cm), jnp.float32),
    )(acc, bias)


@jax.jit
def _finalize3(acc, bias):
    return pl.pallas_call(
        _fin3_body,
        out_shape=jax.ShapeDtypeStruct((NACC, 64), jnp.float32),
    )(acc, bias)


def _leaky(x):
    return jnp.maximum(x, 0.2 * x)


def kernel(x, edge_index, W1, as1, ad1, b1, W2, as2, ad2, b2, W3, as3, ad3, b3):
    cp = jnp.asarray(_COLPERM)

    loop = jnp.arange(N, dtype=jnp.int32)
    padv = jnp.full((ETP - ET,), N, jnp.int32)
    src = jnp.concatenate([edge_index[0], loop, padv]).reshape(NC * NS, NB, B)
    dst = jnp.concatenate([edge_index[1], loop, padv]).reshape(NC * NS, NB, B)
    idx = jnp.stack([src, dst], axis=2)

    rep2 = lambda a: jnp.repeat(a, 2, axis=1)
    wcat1 = jnp.concatenate(
        [W1[:, cp], rep2(_fold_att(W1, as1)), rep2(_fold_att(W1, ad1))], axis=1)
    W2r = W2[cp, :]
    wcat2 = jnp.concatenate(
        [W2r[:, cp], rep2(_fold_att(W2r, as2)), rep2(_fold_att(W2r, ad2))], axis=1)
    W3r = W3[cp, :]
    rep16 = lambda a: jnp.repeat(a, 16, axis=1)
    wcat3 = jnp.concatenate(
        [W3r, rep16(_fold_att(W3r, as3)), rep16(_fold_att(W3r, ad3))], axis=1)
    b1p = b1[cp][None, :]
    b2p = b2[cp][None, :]
    b3p = b3[None, :]

    xp = jnp.pad(x, ((0, NPAD - N), (0, 0)))

    h_tab, a_s, a_d, mraw = _prep(xp, wcat1, 128)
    m16 = _leaky(mraw[0])
    acc = _edge_pass(h_tab, a_s, a_d, m16, idx, 144)
    x1 = jnp.pad(_finalize(acc, b1p), ((0, NPAD - NACC), (0, 0)))

    h_tab, a_s, a_d, mraw = _prep(x1, wcat2, 128)
    m16 = _leaky(mraw[0])
    acc = _edge_pass(h_tab, a_s, a_d, m16, idx, 144)
    x2 = jnp.pad(_finalize(acc, b2p), ((0, NPAD - NACC), (0, 0)))

    h_tab, a_s, a_d, mraw = _prep(x2, wcat3, 64)
    m16 = _leaky(mraw[0])
    acc = _edge_pass(h_tab, a_s, a_d, m16, idx, 80)
    out = _finalize3(acc, b3p)
    return out[:N]


# speedup vs baseline: 1.7331x; 1.2736x over previous
import functools

import jax
import jax.numpy as jnp
import numpy as np
from jax import lax
from jax.experimental import pallas as pl
from jax.experimental.pallas import tpu as pltpu
from jax.experimental.pallas import tpu_sc as plsc

N = 10000
E = 320000
NACC = 10016
NC, NS, LANES = 2, 16, 16
B = 112
ET = E + N
NB = -(-ET // (NC * NS * B))
ETP = NC * NS * NB * B

_COLPERM = np.array(
    [16 * ((c % 16) // 2) + 2 * (c // 16) + (c % 16) % 2 for c in range(128)],
    dtype=np.int32,
)

_R = 2504
_GRID = NACC // _R


def _fold_att(W, att):
    heads, ch = att.shape[1], att.shape[2]
    return jnp.einsum("ihc,hc->ih", W.reshape(W.shape[0], heads, ch), att[0])


def _emit_tables(i, x, cm, h_ref, as_ref, ad_ref, m_ref):
    h_ref[:, :cm] = x[:, :cm]
    h_ref[:, cm:] = jnp.ones((_R, 16), jnp.float32)
    a_s = x[:, cm:cm + 16]
    a_d = x[:, cm + 16:cm + 32]
    as_ref[...] = a_s
    ad_ref[...] = a_d
    m = jnp.max(a_s, axis=0, keepdims=True) + jnp.max(a_d, axis=0, keepdims=True)

    @pl.when(i == 0)
    def _():
        m_ref[...] = m

    @pl.when(i > 0)
    def _():
        m_ref[...] = jnp.maximum(m_ref[...], m)


def _prep_body(cm, x_ref, w_ref, h_ref, as_ref, ad_ref, m_ref):
    i = pl.program_id(0)
    h = jnp.dot(x_ref[...], w_ref[...], preferred_element_type=jnp.float32)
    _emit_tables(i, h, cm, h_ref, as_ref, ad_ref, m_ref)


def _table_outs(cm):
    ct = cm + 16
    out_specs = [
        pl.BlockSpec((_R, ct), lambda i: (i, 0)),
        pl.BlockSpec((_R, 16), lambda i: (i, 0)),
        pl.BlockSpec((_R, 16), lambda i: (i, 0)),
        pl.BlockSpec((1, 16), lambda i: (0, 0)),
    ]
    out_shape = [
        jax.ShapeDtypeStruct((NACC, ct), jnp.float32),
        jax.ShapeDtypeStruct((NACC, 16), jnp.float32),
        jax.ShapeDtypeStruct((NACC, 16), jnp.float32),
        jax.ShapeDtypeStruct((1, 16), jnp.float32),
    ]
    return out_specs, out_shape


@functools.partial(jax.jit, static_argnums=(2,))
def _prep(xp, wcat, cm):
    out_specs, out_shape = _table_outs(cm)
    return pl.pallas_call(
        functools.partial(_prep_body, cm),
        grid=(_GRID,),
        in_specs=[
            pl.BlockSpec((_R, 128), lambda i: (i, 0)),
            pl.BlockSpec((128, cm + 32), lambda i: (0, 0)),
        ],
        out_specs=out_specs,
        out_shape=out_shape,
    )(xp, wcat)


def _mid_body(cm, acc_ref, b_ref, w_ref, h_ref, as_ref, ad_ref, m_ref):
    i = pl.program_id(0)
    a = acc_ref[0] + acc_ref[1]
    den = a[:, 128:144]
    dfull = jnp.concatenate([den] * 8, axis=1)
    o = a[:, :128] / (dfull + 1e-16) + b_ref[...]
    x = jnp.where(o > 0, o, jnp.exp(o) - 1.0)
    h = jnp.dot(x, w_ref[...], preferred_element_type=jnp.float32)
    _emit_tables(i, h, cm, h_ref, as_ref, ad_ref, m_ref)


@functools.partial(jax.jit, static_argnums=(3,))
def _mid(acc, bias, wcat, cm):
    out_specs, out_shape = _table_outs(cm)
    return pl.pallas_call(
        functools.partial(_mid_body, cm),
        grid=(_GRID,),
        in_specs=[
            pl.BlockSpec((2, _R, 144), lambda i: (0, i, 0)),
            pl.BlockSpec((1, 128), lambda i: (0, 0)),
            pl.BlockSpec((128, cm + 32), lambda i: (0, 0)),
        ],
        out_specs=out_specs,
        out_shape=out_shape,
    )(acc, bias, wcat)


def _edge_body(ct, h_hbm, as_hbm, ad_hbm, m_hbm, idx_hbm, out_hbm,
               acc, idx_v, as_v, ad_v, h_v, m_v,
               gsa, gsd, gsh, scs, ixs):
    nq = ct // 16
    c = lax.axis_index("c")
    s = lax.axis_index("s")
    wid = c * NS + s
    rows = NACC // NS

    def g_as(b_slot, i_slot):
        return pltpu.make_async_copy(
            as_hbm.at[idx_v.at[i_slot, 0]], as_v.at[b_slot], gsa.at[b_slot])

    def g_ad(b_slot, i_slot):
        return pltpu.make_async_copy(
            ad_hbm.at[idx_v.at[i_slot, 1]], ad_v.at[b_slot], gsd.at[b_slot])

    def g_h(b_slot, i_slot):
        return pltpu.make_async_copy(
            h_hbm.at[idx_v.at[i_slot, 0]], h_v.at[b_slot], gsh.at[b_slot])

    def sc_start(b_slot, i_slot):
        pltpu.async_copy(h_v.at[b_slot], acc.at[idx_v.at[i_slot, 1]],
                         scs, add=True)

    def sc_wait():
        pltpu.make_async_copy(h_v.at[0], acc.at[idx_v.at[0, 1]], scs).wait()

    def g_idx(blk, i_slot):
        return pltpu.make_async_copy(
            idx_hbm.at[wid, blk], idx_v.at[i_slot], ixs)

    def issue_gather(b_slot, i_slot):
        g_as(b_slot, i_slot).start()
        g_ad(b_slot, i_slot).start()
        g_h(b_slot, i_slot).start()

    def wait_gather(b_slot):
        g_as(b_slot, 0).wait()
        g_ad(b_slot, 0).wait()
        g_h(b_slot, 0).wait()

    def zfill(r, _):
        for q in range(nq):
            h_v[0, r, pl.ds(16 * q, 16)] = jnp.zeros((16,), jnp.float32)
        return 0
    lax.fori_loop(0, B, zfill, 0)

    def zcopy(k, _):
        pltpu.sync_copy(h_v.at[0], acc.at[pl.ds(s * rows + k * B, B)])
        return 0
    lax.fori_loop(0, rows // B, zcopy, 0)
    pltpu.sync_copy(h_v.at[0, pl.ds(0, rows - B * (rows // B))],
                    acc.at[pl.ds(s * rows + B * (rows // B),
                                 rows - B * (rows // B))])

    pltpu.sync_copy(m_hbm, m_v)
    pltpu.sync_copy(idx_hbm.at[wid, 0], idx_v.at[0])
    pltpu.sync_copy(idx_hbm.at[wid, 1], idx_v.at[1])
    g_idx(2, 2).start()
    plsc.subcore_barrier()
    issue_gather(0, 0)
    mraw = m_v[0, :]
    mvec = jnp.maximum(mraw, 0.2 * mraw)

    def blk(b, _):
        p = lax.rem(b, 2)
        pbar = 1 - p
        i1 = lax.rem(b + 1, 3)
        i2 = lax.rem(b + 2, 3)
        wait_gather(p)

        @pl.when(b >= 1)
        def _():
            sc_wait()

        @pl.when(jnp.logical_and(b >= 1, b + 1 < NB))
        def _():
            g_idx(0, 0).wait()

        @pl.when(b + 1 < NB)
        def _():
            issue_gather(pbar, i1)

        @plsc.parallel_loop(0, B, step=1, unroll=4)
        def _(i):
            e = as_v[p, i, :] + ad_v[p, i, :]
            e = jnp.maximum(e, 0.2 * e)
            w = jnp.exp(e - mvec)
            for q in range(nq):
                h_v[p, i, pl.ds(16 * q, 16)] = (
                    h_v[p, i, pl.ds(16 * q, 16)] * w)

        sc_start(p, lax.rem(b, 3))

        @pl.when(jnp.logical_and(b >= 1, b + 2 < NB))
        def _():
            g_idx(b + 2, i2).start()
        return 0
    lax.fori_loop(0, NB, blk, 0)

    sc_wait()
    plsc.subcore_barrier()
    pltpu.sync_copy(acc.at[pl.ds(s * rows, rows)],
                    out_hbm.at[c, pl.ds(s * rows, rows)])


@functools.partial(jax.jit, static_argnums=(5,))
def _edge_pass(h_tab, as_tab, ad_tab, m16, idx, ct):
    mesh = plsc.VectorSubcoreMesh(
        core_axis_name="c", subcore_axis_name="s",
        num_cores=NC, num_subcores=NS)
    return pl.kernel(
        functools.partial(_edge_body, ct),
        out_type=jax.ShapeDtypeStruct((NC, NACC, ct), jnp.float32),
        mesh=mesh,
        compiler_params=pltpu.CompilerParams(use_tc_tiling_on_sc=False),
        scratch_types=[
            pltpu.VMEM_SHARED((NACC, ct), jnp.float32),
            pltpu.VMEM((3, 2, B), jnp.int32),
            pltpu.VMEM((2, B, 16), jnp.float32),
            pltpu.VMEM((2, B, 16), jnp.float32),
            pltpu.VMEM((2, B, ct), jnp.float32),
            pltpu.VMEM((1, 16), jnp.float32),
            pltpu.SemaphoreType.DMA((2,)),
            pltpu.SemaphoreType.DMA((2,)),
            pltpu.SemaphoreType.DMA((2,)),
            pltpu.SemaphoreType.DMA,
            pltpu.SemaphoreType.DMA,
        ],
    )(h_tab, as_tab, ad_tab, m16, idx)


def _fin3_body(acc_ref, b_ref, o_ref):
    a = acc_ref[0] + acc_ref[1]
    den = a[:, 64:80]
    dfull = jnp.concatenate([den] * 4, axis=1)
    o = a[:, :64] / (dfull + 1e-16) + b_ref[...]
    m = jnp.max(o, axis=1, keepdims=True)
    z = o - m
    o_ref[...] = z - jnp.log(jnp.sum(jnp.exp(z), axis=1, keepdims=True))


@jax.jit
def _finalize3(acc, bias):
    return pl.pallas_call(
        _fin3_body,
        out_shape=jax.ShapeDtypeStruct((NACC, 64), jnp.float32),
    )(acc, bias)


def kernel(x, edge_index, W1, as1, ad1, b1, W2, as2, ad2, b2, W3, as3, ad3, b3):
    cp = jnp.asarray(_COLPERM)

    loop = jnp.arange(N, dtype=jnp.int32)
    padv = jnp.full((ETP - ET,), N, jnp.int32)
    src = jnp.concatenate([edge_index[0], loop, padv]).reshape(NC * NS, NB, B)
    dst = jnp.concatenate([edge_index[1], loop, padv]).reshape(NC * NS, NB, B)
    idx = jnp.stack([src, dst], axis=2)

    rep2 = lambda a: jnp.repeat(a, 2, axis=1)
    wcat1 = jnp.concatenate(
        [W1[:, cp], rep2(_fold_att(W1, as1)), rep2(_fold_att(W1, ad1))], axis=1)
    W2r = W2[cp, :]
    wcat2 = jnp.concatenate(
        [W2r[:, cp], rep2(_fold_att(W2r, as2)), rep2(_fold_att(W2r, ad2))], axis=1)
    W3r = W3[cp, :]
    rep16 = lambda a: jnp.repeat(a, 16, axis=1)
    wcat3 = jnp.concatenate(
        [W3r, rep16(_fold_att(W3r, as3)), rep16(_fold_att(W3r, ad3))], axis=1)
    b1p = b1[cp][None, :]
    b2p = b2[cp][None, :]
    b3p = b3[None, :]

    xp = jnp.pad(x, ((0, NACC - N), (0, 0)))

    h_tab, a_s, a_d, mraw = _prep(xp, wcat1, 128)
    acc = _edge_pass(h_tab, a_s, a_d, mraw, idx, 144)
    h_tab, a_s, a_d, mraw = _mid(acc, b1p, wcat2, 128)
    acc = _edge_pass(h_tab, a_s, a_d, mraw, idx, 144)
    h_tab, a_s, a_d, mraw = _mid(acc, b2p, wcat3, 64)
    acc = _edge_pass(h_tab, a_s, a_d, mraw, idx, 80)
    out = _finalize3(acc, b3p)
    return out[:N]


# speedup vs baseline: 1.8186x; 1.0494x over previous
import functools

import jax
import jax.numpy as jnp
import numpy as np
from jax import lax
from jax.experimental import pallas as pl
from jax.experimental.pallas import tpu as pltpu
from jax.experimental.pallas import tpu_sc as plsc

N = 10000
E = 320000
NACC = 10016
NC, NS, LANES = 2, 16, 16
B = 112
ET = E + N
NB = -(-ET // (NC * NS * B))
ETP = NC * NS * NB * B

_COLPERM = np.array(
    [16 * ((c % 16) // 2) + 2 * (c // 16) + (c % 16) % 2 for c in range(128)],
    dtype=np.int32,
)

_R = 2504
_GRID = NACC // _R


def _fold_att(W, att):
    heads, ch = att.shape[1], att.shape[2]
    return jnp.einsum("ihc,hc->ih", W.reshape(W.shape[0], heads, ch), att[0])


def _emit_tables(i, x, cm, h_ref, as_ref, ad_ref, m_ref):
    h_ref[:, :cm] = x[:, :cm]
    h_ref[:, cm:] = jnp.ones((_R, 16), jnp.float32)
    a_s = x[:, cm:cm + 16]
    a_d = x[:, cm + 16:cm + 32]
    as_ref[...] = a_s
    ad_ref[...] = a_d
    m = jnp.max(a_s, axis=0, keepdims=True) + jnp.max(a_d, axis=0, keepdims=True)

    @pl.when(i == 0)
    def _():
        m_ref[...] = m

    @pl.when(i > 0)
    def _():
        m_ref[...] = jnp.maximum(m_ref[...], m)


def _prep_body(cm, x_ref, w_ref, h_ref, as_ref, ad_ref, m_ref):
    i = pl.program_id(0)
    h = jnp.dot(x_ref[...], w_ref[...], preferred_element_type=jnp.float32)
    _emit_tables(i, h, cm, h_ref, as_ref, ad_ref, m_ref)


def _table_outs(cm):
    ct = cm + 16
    out_specs = [
        pl.BlockSpec((_R, ct), lambda i: (i, 0)),
        pl.BlockSpec((_R, 16), lambda i: (i, 0)),
        pl.BlockSpec((_R, 16), lambda i: (i, 0)),
        pl.BlockSpec((1, 16), lambda i: (0, 0)),
    ]
    out_shape = [
        jax.ShapeDtypeStruct((NACC, ct), jnp.float32),
        jax.ShapeDtypeStruct((NACC, 16), jnp.float32),
        jax.ShapeDtypeStruct((NACC, 16), jnp.float32),
        jax.ShapeDtypeStruct((1, 16), jnp.float32),
    ]
    return out_specs, out_shape


@functools.partial(jax.jit, static_argnums=(2,))
def _prep(xp, wcat, cm):
    out_specs, out_shape = _table_outs(cm)
    return pl.pallas_call(
        functools.partial(_prep_body, cm),
        grid=(_GRID,),
        in_specs=[
            pl.BlockSpec((_R, 128), lambda i: (i, 0)),
            pl.BlockSpec((128, cm + 32), lambda i: (0, 0)),
        ],
        out_specs=out_specs,
        out_shape=out_shape,
    )(xp, wcat)


def _mid_body(cm, acc_ref, b_ref, w_ref, h_ref, as_ref, ad_ref, m_ref):
    i = pl.program_id(0)
    a = acc_ref[0] + acc_ref[1]
    den = a[:, 128:144]
    dfull = jnp.concatenate([den] * 8, axis=1)
    o = a[:, :128] / (dfull + 1e-16) + b_ref[...]
    x = jnp.where(o > 0, o, jnp.exp(o) - 1.0)
    h = jnp.dot(x, w_ref[...], preferred_element_type=jnp.float32)
    _emit_tables(i, h, cm, h_ref, as_ref, ad_ref, m_ref)


@functools.partial(jax.jit, static_argnums=(3,))
def _mid(acc, bias, wcat, cm):
    out_specs, out_shape = _table_outs(cm)
    return pl.pallas_call(
        functools.partial(_mid_body, cm),
        grid=(_GRID,),
        in_specs=[
            pl.BlockSpec((2, _R, 144), lambda i: (0, i, 0)),
            pl.BlockSpec((1, 128), lambda i: (0, 0)),
            pl.BlockSpec((128, cm + 32), lambda i: (0, 0)),
        ],
        out_specs=out_specs,
        out_shape=out_shape,
    )(acc, bias, wcat)


def _edge_body(ct, h_hbm, as_hbm, ad_hbm, m_hbm, idx_hbm, out_hbm,
               acc, idx_v, as_v, ad_v, h_v, m_v,
               gsa, gsd, gsh, scs, ixs):
    nq = ct // 16
    c = lax.axis_index("c")
    s = lax.axis_index("s")
    wid = c * NS + s
    rows = NACC // NS

    def g_as(b_slot, i_slot):
        return pltpu.make_async_copy(
            as_hbm.at[idx_v.at[i_slot, 0]], as_v.at[b_slot], gsa.at[b_slot])

    def g_ad(b_slot, i_slot):
        return pltpu.make_async_copy(
            ad_hbm.at[idx_v.at[i_slot, 1]], ad_v.at[b_slot], gsd.at[b_slot])

    def g_h(b_slot, i_slot):
        return pltpu.make_async_copy(
            h_hbm.at[idx_v.at[i_slot, 0]], h_v.at[b_slot], gsh.at[b_slot])

    def sc_start(b_slot, i_slot):
        pltpu.async_copy(h_v.at[b_slot], acc.at[idx_v.at[i_slot, 1]],
                         scs, add=True)

    def sc_wait():
        pltpu.make_async_copy(h_v.at[0], acc.at[idx_v.at[0, 1]], scs).wait()

    def g_idx(blk, i_slot):
        return pltpu.make_async_copy(
            idx_hbm.at[wid, blk], idx_v.at[i_slot], ixs)

    def issue_gather(b_slot, i_slot):
        g_as(b_slot, i_slot).start()
        g_ad(b_slot, i_slot).start()
        g_h(b_slot, i_slot).start()

    def wait_gather(b_slot):
        g_as(b_slot, 0).wait()
        g_ad(b_slot, 0).wait()
        g_h(b_slot, 0).wait()

    def zfill(r, _):
        for q in range(nq):
            h_v[0, r, pl.ds(16 * q, 16)] = jnp.zeros((16,), jnp.float32)
        return 0
    lax.fori_loop(0, B, zfill, 0)

    def zcopy(k, _):
        pltpu.sync_copy(h_v.at[0], acc.at[pl.ds(s * rows + k * B, B)])
        return 0
    lax.fori_loop(0, rows // B, zcopy, 0)
    pltpu.sync_copy(h_v.at[0, pl.ds(0, rows - B * (rows // B))],
                    acc.at[pl.ds(s * rows + B * (rows // B),
                                 rows - B * (rows // B))])

    pltpu.sync_copy(m_hbm, m_v)
    pltpu.sync_copy(idx_hbm.at[wid, 0], idx_v.at[0])
    pltpu.sync_copy(idx_hbm.at[wid, 1], idx_v.at[1])
    g_idx(2, 2).start()
    plsc.subcore_barrier()
    issue_gather(0, 0)
    mraw = m_v[0, :]
    mvec = jnp.maximum(mraw, 0.2 * mraw)

    def blk(b, _):
        p = lax.rem(b, 2)
        pbar = 1 - p
        i1 = lax.rem(b + 1, 3)
        i2 = lax.rem(b + 2, 3)
        wait_gather(p)

        @pl.when(b >= 1)
        def _():
            sc_wait()

        @pl.when(jnp.logical_and(b >= 1, b + 1 < NB))
        def _():
            g_idx(0, 0).wait()

        @pl.when(b + 1 < NB)
        def _():
            issue_gather(pbar, i1)

        @plsc.parallel_loop(0, B, step=1, unroll=4)
        def _(i):
            e = as_v[p, i, :] + ad_v[p, i, :]
            e = jnp.maximum(e, 0.2 * e)
            w = jnp.exp(e - mvec)
            for q in range(nq):
                h_v[p, i, pl.ds(16 * q, 16)] = (
                    h_v[p, i, pl.ds(16 * q, 16)] * w)

        sc_start(p, lax.rem(b, 3))

        @pl.when(jnp.logical_and(b >= 1, b + 2 < NB))
        def _():
            g_idx(b + 2, i2).start()
        return 0
    lax.fori_loop(0, NB, blk, 0)

    sc_wait()
    plsc.subcore_barrier()
    pltpu.sync_copy(acc.at[pl.ds(s * rows, rows)],
                    out_hbm.at[c, pl.ds(s * rows, rows)])


@functools.partial(jax.jit, static_argnums=(5,))
def _edge_pass(h_tab, as_tab, ad_tab, m16, idx, ct):
    mesh = plsc.VectorSubcoreMesh(
        core_axis_name="c", subcore_axis_name="s",
        num_cores=NC, num_subcores=NS)
    return pl.kernel(
        functools.partial(_edge_body, ct),
        out_type=jax.ShapeDtypeStruct((NC, NACC, ct), jnp.float32),
        mesh=mesh,
        compiler_params=pltpu.CompilerParams(use_tc_tiling_on_sc=False),
        scratch_types=[
            pltpu.VMEM_SHARED((NACC, ct), jnp.float32),
            pltpu.VMEM((3, 2, B), jnp.int32),
            pltpu.VMEM((2, B, 16), jnp.float32),
            pltpu.VMEM((2, B, 16), jnp.float32),
            pltpu.VMEM((2, B, ct), jnp.float32),
            pltpu.VMEM((1, 16), jnp.float32),
            pltpu.SemaphoreType.DMA((2,)),
            pltpu.SemaphoreType.DMA((2,)),
            pltpu.SemaphoreType.DMA((2,)),
            pltpu.SemaphoreType.DMA,
            pltpu.SemaphoreType.DMA,
        ],
    )(h_tab, as_tab, ad_tab, m16, idx)


def _fin3_body(acc_ref, b_ref, o_ref):
    a = acc_ref[0] + acc_ref[1]
    den = a[:, 64:80]
    dfull = jnp.concatenate([den] * 4, axis=1)
    o = a[:, :64] / (dfull + 1e-16) + b_ref[...]
    m = jnp.max(o, axis=1, keepdims=True)
    z = o - m
    o_ref[...] = z - jnp.log(jnp.sum(jnp.exp(z), axis=1, keepdims=True))


@jax.jit
def _finalize3(acc, bias):
    return pl.pallas_call(
        _fin3_body,
        out_shape=jax.ShapeDtypeStruct((NACC, 64), jnp.float32),
    )(acc, bias)


def kernel(x, edge_index, W1, as1, ad1, b1, W2, as2, ad2, b2, W3, as3, ad3, b3):
    cp = jnp.asarray(_COLPERM)

    loop = jnp.arange(N, dtype=jnp.int32)
    padv = jnp.full((ETP - ET,), N, jnp.int32)
    src = jnp.concatenate([edge_index[0], loop, padv]).reshape(NC * NS, NB, B)
    dst = jnp.concatenate([edge_index[1], loop, padv]).reshape(NC * NS, NB, B)
    idx = jnp.stack([src, dst], axis=2)
    idx = jnp.concatenate([idx[16:], idx[:16]])

    rep2 = lambda a: jnp.repeat(a, 2, axis=1)
    wcat1 = jnp.concatenate(
        [W1[:, cp], rep2(_fold_att(W1, as1)), rep2(_fold_att(W1, ad1))], axis=1)
    W2r = W2[cp, :]
    wcat2 = jnp.concatenate(
        [W2r[:, cp], rep2(_fold_att(W2r, as2)), rep2(_fold_att(W2r, ad2))], axis=1)
    W3r = W3[cp, :]
    rep16 = lambda a: jnp.repeat(a, 16, axis=1)
    wcat3 = jnp.concatenate(
        [W3r, rep16(_fold_att(W3r, as3)), rep16(_fold_att(W3r, ad3))], axis=1)
    b1p = b1[cp][None, :]
    b2p = b2[cp][None, :]
    b3p = b3[None, :]

    xp = jnp.pad(x, ((0, NACC - N), (0, 0)))

    h_tab, a_s, a_d, mraw = _prep(xp, wcat1, 128)
    acc = _edge_pass(h_tab, a_s, a_d, mraw, idx, 144)
    h_tab, a_s, a_d, mraw = _mid(acc, b1p, wcat2, 128)
    acc = _edge_pass(h_tab, a_s, a_d, mraw, idx, 144)
    h_tab, a_s, a_d, mraw = _mid(acc, b2p, wcat3, 64)
    acc = _edge_pass(h_tab, a_s, a_d, mraw, idx, 80)
    out = _finalize3(acc, b3p)
    return out[:N]
